# baseline (speedup 1.0000x reference)
import functools

import jax
import jax.numpy as jnp
from jax import lax
from jax.experimental import pallas as pl
from jax.experimental.pallas import tpu as pltpu
from jax.experimental.pallas import tpu_sc as plsc

N = 16384
M = 16384
K = 16
B = 64
NW = 32
RPW = N // NW
CR = 32
NCH = RPW // CR
NI = CR * K
GB = 4
LB = 16


def _body(xT_hbm, w_hbm, bias_hbm, idx_hbm, out_hbm,
          idx_v, g_v0, g_v1, w_v, b_v, o_v0, o_v1,
          sem_g0, sem_g1, sem_o0, sem_o1):
    wid = lax.axis_index("s") * 2 + lax.axis_index("c")
    row0 = wid * RPW
    g_bufs = (g_v0, g_v1)
    o_bufs = (o_v0, o_v1)
    sem_gs = (sem_g0, sem_g1)
    sem_os = (sem_o0, sem_o1)

    pltpu.sync_copy(idx_hbm.at[pl.ds(wid * NCH, NCH)], idx_v)
    pltpu.sync_copy(w_hbm.at[pl.ds(row0 * K, RPW * K)], w_v)
    pltpu.sync_copy(bias_hbm.at[pl.ds(row0, RPW)], b_v)

    def gathers(c, h):
        for j in range(GB):
            pltpu.async_copy(xT_hbm.at[idx_v.at[c, j]], g_bufs[h].at[j],
                             sem_gs[h])

    def half_chunk(c, h):
        g_v, o_v, sem_o = g_bufs[h], o_bufs[h], sem_os[h]

        @pl.when(c + 1 < NCH)
        def _():
            gathers(c + 1, 1 - h)

        @pl.when(c >= 2)
        def _():
            pltpu.make_async_copy(o_v, out_hbm.at[pl.ds(row0, CR)],
                                  sem_o).wait()

        for j in range(GB):
            pltpu.make_async_copy(xT_hbm.at[idx_v.at[0, j]], g_v.at[j],
                                  sem_gs[h]).wait()

        def row_group(g, _):
            bgrp = b_v[pl.ds(c * CR + g * LB, LB)]
            for l in range(LB):
                wrow = w_v[pl.ds(c * NI + g * 256 + l * K, K)]
                blk = g * 2 + (l // 8)
                r0 = (l % 8) * K
                accs = [jnp.full((LB,), bgrp[l], dtype=jnp.float32)
                        for _ in range(B // LB)]
                for k in range(K):
                    wv = jnp.full((LB,), wrow[k], dtype=jnp.float32)
                    for j in range(B // LB):
                        accs[j] = accs[j] + wv * g_v[blk, r0 + k,
                                                     pl.ds(j * LB, LB)]
                for j in range(B // LB):
                    o_v[g * LB + l, pl.ds(j * LB, LB)] = accs[j]
            return ()

        lax.fori_loop(0, CR // LB, row_group, (), unroll=False)
        pltpu.async_copy(o_v, out_hbm.at[pl.ds(row0 + c * CR, CR)], sem_o)

    gathers(0, 0)

    def chunk_pair(dc, _):
        half_chunk(dc * 2, 0)
        half_chunk(dc * 2 + 1, 1)
        return ()

    lax.fori_loop(0, NCH // 2, chunk_pair, (), unroll=False)
    for h in range(2):
        pltpu.make_async_copy(o_bufs[h], out_hbm.at[pl.ds(row0, CR)],
                              sem_os[h]).wait()


@jax.jit
def _spmm(xT, W_val, bias, idx3):
    mesh = plsc.VectorSubcoreMesh(core_axis_name="c", subcore_axis_name="s")
    f = pl.kernel(
        _body,
        out_type=jax.ShapeDtypeStruct((N, B), jnp.float32),
        mesh=mesh,
        scratch_types=[
            pltpu.VMEM((NCH, GB, 128), jnp.int32),
            pltpu.VMEM((GB, 128, B), jnp.float32),
            pltpu.VMEM((GB, 128, B), jnp.float32),
            pltpu.VMEM((RPW * K,), jnp.float32),
            pltpu.VMEM((RPW,), jnp.float32),
            pltpu.VMEM((CR, B), jnp.float32),
            pltpu.VMEM((CR, B), jnp.float32),
            pltpu.SemaphoreType.DMA,
            pltpu.SemaphoreType.DMA,
            pltpu.SemaphoreType.DMA,
            pltpu.SemaphoreType.DMA,
        ],
        compiler_params=pltpu.CompilerParams(use_tc_tiling_on_sc=False),
    )
    return f(xT, W_val, bias, idx3)


def kernel(input, W_val, bias, indices, rows):
    x2 = input.reshape(-1, input.shape[-1])
    xT = x2.T
    idx3 = indices.reshape(-1, GB, 128)
    yT = _spmm(xT, W_val, bias, idx3)
    return yT.T.reshape(input.shape[:-1] + (N,))


# speedup vs baseline: 1.1958x; 1.1958x over previous
import functools

import jax
import jax.numpy as jnp
from jax import lax
from jax.experimental import pallas as pl
from jax.experimental.pallas import tpu as pltpu
from jax.experimental.pallas import tpu_sc as plsc

N = 16384
M = 16384
K = 16
B = 64
NW = 32
RPW = N // NW
CR = 32
NCH = RPW // CR
NI = CR * K
GB = 4
LB = 16
WBK = 4


def _body(xT_hbm, w_hbm, bias_hbm, idx_hbm, out_hbm,
          idx_v, g_v, w_v, b_v, o_t, sem_g, sem_o):
    wid = lax.axis_index("s") * 2 + lax.axis_index("c")
    row0 = wid * RPW

    pltpu.sync_copy(idx_hbm.at[pl.ds(wid * NCH, NCH)], idx_v)
    pltpu.sync_copy(w_hbm.at[pl.ds(row0 * K, RPW * K)], w_v)
    pltpu.sync_copy(bias_hbm.at[pl.ds(row0, RPW)], b_v)

    def gathers(c, p):
        for j in range(GB):
            pltpu.async_copy(xT_hbm.at[idx_v.at[c, j]], g_v.at[p, j],
                             sem_g.at[p])

    gathers(0, 0)
    bidxs = [lax.iota(jnp.int32, LB) + (j * LB) for j in range(B // LB)]

    def chunk(c, _):
        p = lax.rem(c, 2)

        @pl.when(c + 1 < NCH)
        def _():
            gathers(c + 1, 1 - p)

        for j in range(GB):
            pltpu.make_async_copy(xT_hbm.at[idx_v.at[0, j]], g_v.at[p, j],
                                  sem_g.at[p]).wait()

        def row_group(g, _):
            bgrp = b_v[pl.ds(c * CR + g * LB, LB)]
            for l in range(LB):
                wrow = w_v[pl.ds(c * NI + g * 256 + l * K, K)]
                blk = g * 2 + (l // 8)
                r0 = (l % 8) * K
                accs = [jnp.full((LB,), bgrp[l], dtype=jnp.float32)
                        for _ in range(B // LB)]
                for k in range(K):
                    wv = jnp.full((LB,), wrow[k], dtype=jnp.float32)
                    for j in range(B // LB):
                        accs[j] = accs[j] + wv * g_v[p, blk, r0 + k,
                                                     pl.ds(j * LB, LB)]
                rvec = jnp.full((LB,), c * CR + g * LB + l, dtype=jnp.int32)
                for j in range(B // LB):
                    plsc.store_scatter(o_t, [bidxs[j], rvec], accs[j])
            return ()

        lax.fori_loop(0, CR // LB, row_group, (), unroll=False)

        @pl.when(lax.rem(c, NCH // WBK) == (NCH // WBK - 1))
        def _():
            q = c // (NCH // WBK)
            cols = RPW // WBK
            pltpu.async_copy(
                o_t.at[:, pl.ds(q * cols, cols)],
                out_hbm.at[:, pl.ds(row0 + q * cols, cols)], sem_o)
        return ()

    lax.fori_loop(0, NCH, chunk, (), unroll=False)
    for q in range(WBK):
        cols = RPW // WBK
        pltpu.make_async_copy(o_t.at[:, pl.ds(q * cols, cols)],
                              out_hbm.at[:, pl.ds(row0, cols)], sem_o).wait()


@jax.jit
def _spmm(xT, W_val, bias, idx3):
    mesh = plsc.VectorSubcoreMesh(core_axis_name="c", subcore_axis_name="s")
    f = pl.kernel(
        _body,
        out_type=jax.ShapeDtypeStruct((B, N), jnp.float32),
        mesh=mesh,
        scratch_types=[
            pltpu.VMEM((NCH, GB, 128), jnp.int32),
            pltpu.VMEM((2, GB, 128, B), jnp.float32),
            pltpu.VMEM((RPW * K,), jnp.float32),
            pltpu.VMEM((RPW,), jnp.float32),
            pltpu.VMEM((B, RPW), jnp.float32),
            pltpu.SemaphoreType.DMA((2,)),
            pltpu.SemaphoreType.DMA,
        ],
        compiler_params=pltpu.CompilerParams(use_tc_tiling_on_sc=False,
                                             needs_layout_passes=False),
    )
    return f(xT, W_val, bias, idx3)


def kernel(input, W_val, bias, indices, rows):
    x2 = input.reshape(-1, input.shape[-1])
    xT = x2.T
    idx3 = indices.reshape(-1, GB, 128)
    y = _spmm(xT, W_val, bias, idx3)
    return y.reshape(input.shape[:-1] + (N,))


# speedup vs baseline: 1.3134x; 1.0984x over previous
import functools

import jax
import jax.numpy as jnp
from jax import lax
from jax.experimental import pallas as pl
from jax.experimental.pallas import tpu as pltpu
from jax.experimental.pallas import tpu_sc as plsc

N = 16384
M = 16384
K = 16
B = 64
NW = 32
RPW = N // NW
CR = 32
NCH = RPW // CR
NI = CR * K
GB = 4
LB = 16


def _body(xT_hbm, w_hbm, bias_hbm, idx_hbm, out_hbm,
          idx_v, g_v, w_v, b_v, o_v, sem_ga, sem_gb, sem_o):
    wid = lax.axis_index("s") * 2 + lax.axis_index("c")
    row0 = wid * RPW

    pltpu.sync_copy(idx_hbm.at[pl.ds(wid * NCH, NCH)], idx_v)
    pltpu.sync_copy(w_hbm.at[pl.ds(row0 * K, RPW * K)], w_v)
    pltpu.sync_copy(bias_hbm.at[pl.ds(row0, RPW)], b_v)

    def gathers(c, p):
        for j in range(GB):
            sem = sem_ga if j < 2 else sem_gb
            pltpu.async_copy(xT_hbm.at[idx_v.at[c, j]], g_v.at[p, j],
                             sem.at[p])

    def drain(sem, p, nblocks):
        for _ in range(nblocks):
            pltpu.make_async_copy(xT_hbm.at[idx_v.at[0, 0]], g_v.at[p, 0],
                                  sem.at[p]).wait()

    gathers(0, 0)

    def chunk(c, _):
        p = lax.rem(c, 2)

        @pl.when(c + 1 < NCH)
        def _():
            gathers(c + 1, 1 - p)

        @pl.when(c >= 2)
        def _():
            pltpu.make_async_copy(o_v.at[p], out_hbm.at[pl.ds(row0, CR)],
                                  sem_o.at[p]).wait()

        def row_group(g, _):
            @pl.when(g == 0)
            def _():
                drain(sem_ga, p, 2)

            @pl.when(g == 1)
            def _():
                drain(sem_gb, p, 2)

            bgrp = b_v[pl.ds(c * CR + g * LB, LB)]
            for l in range(LB):
                wrow = w_v[pl.ds(c * NI + g * 256 + l * K, K)]
                blk = g * 2 + (l // 8)
                r0 = (l % 8) * K
                accs = [jnp.full((LB,), bgrp[l], dtype=jnp.float32)
                        for _ in range(B // LB)]
                for k in range(K):
                    wv = jnp.full((LB,), wrow[k], dtype=jnp.float32)
                    for j in range(B // LB):
                        accs[j] = accs[j] + wv * g_v[p, blk, r0 + k,
                                                     pl.ds(j * LB, LB)]
                for j in range(B // LB):
                    o_v[p, g * LB + l, pl.ds(j * LB, LB)] = accs[j]
            return ()

        lax.fori_loop(0, CR // LB, row_group, (), unroll=False)
        pltpu.async_copy(o_v.at[p],
                         out_hbm.at[pl.ds(row0 + c * CR, CR)], sem_o.at[p])
        return ()

    lax.fori_loop(0, NCH, chunk, (), unroll=False)
    for p in range(2):
        pltpu.make_async_copy(o_v.at[p], out_hbm.at[pl.ds(row0, CR)],
                              sem_o.at[p]).wait()


@jax.jit
def _spmm(xT, W_val, bias, idx3):
    mesh = plsc.VectorSubcoreMesh(core_axis_name="c", subcore_axis_name="s")
    f = pl.kernel(
        _body,
        out_type=jax.ShapeDtypeStruct((N, B), jnp.float32),
        mesh=mesh,
        scratch_types=[
            pltpu.VMEM((NCH, GB, 128), jnp.int32),
            pltpu.VMEM((2, GB, 128, B), jnp.float32),
            pltpu.VMEM((RPW * K,), jnp.float32),
            pltpu.VMEM((RPW,), jnp.float32),
            pltpu.VMEM((2, CR, B), jnp.float32),
            pltpu.SemaphoreType.DMA((2,)),
            pltpu.SemaphoreType.DMA((2,)),
            pltpu.SemaphoreType.DMA((2,)),
        ],
        compiler_params=pltpu.CompilerParams(use_tc_tiling_on_sc=False),
    )
    return f(xT, W_val, bias, idx3)


def kernel(input, W_val, bias, indices, rows):
    x2 = input.reshape(-1, input.shape[-1])
    xT = x2.T
    idx3 = indices.reshape(-1, GB, 128)
    yT = _spmm(xT, W_val, bias, idx3)
    return yT.T.reshape(input.shape[:-1] + (N,))
